# baseline (speedup 1.0000x reference)
import functools

import jax
import jax.numpy as jnp
from jax import lax
from jax.experimental import pallas as pl
from jax.experimental.pallas import tpu as pltpu
from jax.experimental.pallas import tpu_sc as plsc

N = 10000
E = 160000
H = 256
G = 64
P = 32
MAX_Z = 101
NBT = 5
L = 2

NP = 10240
NB = NP // 256
HH = H // 2
NSUB = 16
EPC = 128
CHUNKS = 80
EPAD = NSUB * CHUNKS * EPC
RPS = NP // NSUB
DUMMY_DST = N + 128

_HIGH = lax.Precision.HIGHEST


def _dot(a, b):
  return jnp.dot(a, b, preferred_element_type=jnp.float32)


def _dot_t(a, b):
  return lax.dot_general(a, b, (((0,), (0,)), ((), ())),
                         preferred_element_type=jnp.float32, precision=_HIGH)


def _ytab_part(nh, be, w, b, y0_ref, y1_ref):
  e5 = _dot(be, w) + b[0:1]
  for t in range(NBT):
    y = jax.nn.relu(nh + e5[t:t + 1])
    y0_ref[t] = y[:, :HH]
    y1_ref[t] = y[:, HH:]


def _mlp_part(nh_ref, a0_ref, a1_ref, w1_ref, b1_ref, w2_ref, b2_ref, g_ref,
              lb_ref):
  ag = jnp.concatenate([a0_ref[...], a1_ref[...]], axis=-1)
  h = nh_ref[...] + ag
  t1 = jax.nn.relu(_dot(h, w1_ref[...]) + b1_ref[0:1])
  t2 = _dot(t1, w2_ref[...]) + b2_ref[0:1]
  mu = jnp.mean(t2, axis=-1, keepdims=True)
  var = jnp.mean((t2 - mu) ** 2, axis=-1, keepdims=True)
  hn = (t2 - mu) / jnp.sqrt(var + 1e-5) * g_ref[0:1] + lb_ref[0:1]
  return jax.nn.relu(hn)


_YTAB_SPECS = [
    pl.BlockSpec((8, H), lambda i: (0, 0)),
    pl.BlockSpec((H, H), lambda i: (0, 0)),
    pl.BlockSpec((8, H), lambda i: (0, 0)),
]
_YTAB_OUT_SPECS = [
    pl.BlockSpec((NBT, 256, HH), lambda i: (0, i, 0)),
    pl.BlockSpec((NBT, 256, HH), lambda i: (0, i, 0)),
]
_YTAB_OUT_SHAPES = [
    jax.ShapeDtypeStruct((NBT, NP, HH), jnp.float32),
    jax.ShapeDtypeStruct((NBT, NP, HH), jnp.float32),
]


def _inityt_body(z_ref, x_ref, ae_ref, wx_ref, bx_ref, be_ref, w_ref, b_ref,
                 nh_ref, y0_ref, y1_ref):
  ohT = (lax.broadcasted_iota(jnp.int32, (128, 256), 0)
         == z_ref[0]).astype(jnp.float32)
  nh = _dot_t(ohT, ae_ref[...]) + _dot(x_ref[...], wx_ref[...]) + bx_ref[0:1]
  nh = jax.nn.relu(nh)
  nh_ref[...] = nh
  _ytab_part(nh, be_ref[...], w_ref[...], b_ref[...], y0_ref, y1_ref)


def _inityt_call(z3, xp, aep, wxp, bx8, bep, w, b8):
  return pl.pallas_call(
      _inityt_body,
      grid=(NB,),
      in_specs=[
          pl.BlockSpec((1, 1, 256), lambda i: (i, 0, 0)),
          pl.BlockSpec((256, 128), lambda i: (i, 0)),
          pl.BlockSpec((128, H), lambda i: (0, 0)),
          pl.BlockSpec((128, H), lambda i: (0, 0)),
          pl.BlockSpec((8, H), lambda i: (0, 0)),
      ] + _YTAB_SPECS,
      out_specs=[pl.BlockSpec((256, H), lambda i: (i, 0))] + _YTAB_OUT_SPECS,
      out_shape=[jax.ShapeDtypeStruct((NP, H), jnp.float32)]
      + _YTAB_OUT_SHAPES,
  )(z3, xp, aep, wxp, bx8, bep, w, b8)


_MLP_SPECS = [
    pl.BlockSpec((256, H), lambda i: (i, 0)),
    pl.BlockSpec((256, HH), lambda i: (i, 0)),
    pl.BlockSpec((256, HH), lambda i: (i, 0)),
    pl.BlockSpec((H, H), lambda i: (0, 0)),
    pl.BlockSpec((8, H), lambda i: (0, 0)),
    pl.BlockSpec((H, H), lambda i: (0, 0)),
    pl.BlockSpec((8, H), lambda i: (0, 0)),
    pl.BlockSpec((8, H), lambda i: (0, 0)),
    pl.BlockSpec((8, H), lambda i: (0, 0)),
]


def _mlpyt_body(nh_ref, a0_ref, a1_ref, w1_ref, b1_ref, w2_ref, b2_ref,
                g_ref, lb_ref, be_ref, w_ref, b_ref, nh_o, y0_ref, y1_ref):
  nh = _mlp_part(nh_ref, a0_ref, a1_ref, w1_ref, b1_ref, w2_ref, b2_ref,
                 g_ref, lb_ref)
  nh_o[...] = nh
  _ytab_part(nh, be_ref[...], w_ref[...], b_ref[...], y0_ref, y1_ref)


def _mlpyt_call(node_h, a0, a1, w1, b18, w2, b28, g8, lb8, bep, w, b8):
  return pl.pallas_call(
      _mlpyt_body,
      grid=(NB,),
      in_specs=_MLP_SPECS + _YTAB_SPECS,
      out_specs=[pl.BlockSpec((256, H), lambda i: (i, 0))] + _YTAB_OUT_SPECS,
      out_shape=[jax.ShapeDtypeStruct((NP, H), jnp.float32)]
      + _YTAB_OUT_SHAPES,
  )(node_h, a0, a1, w1, b18, w2, b28, g8, lb8, bep, w, b8)


def _mlphead_body(nh_ref, a0_ref, a1_ref, w1_ref, b1_ref, w2_ref, b2_ref,
                  g_ref, lb_ref, b3_ref, pp_ref, wp1_ref, bp1_ref, wp2_ref,
                  bp2_ref, whb_ref, wh1_ref, bh1_ref, wh2_ref, bh2_ref,
                  o_ref):
  nh = _mlp_part(nh_ref, a0_ref, a1_ref, w1_ref, b1_ref, w2_ref, b2_ref,
                 g_ref, lb_ref)
  c1 = jax.nn.relu(_dot(pp_ref[...], wp1_ref[...]) + bp1_ref[0:1])
  ctx = _dot(c1, wp2_ref[...]) + bp2_ref[0:1]
  cw = _dot(ctx, whb_ref[...])
  ohT = (lax.broadcasted_iota(jnp.int32, (128, 256), 0)
         == b3_ref[0]).astype(jnp.float32)
  f = jax.nn.relu(_dot(nh, wh1_ref[...]) + _dot_t(ohT, cw) + bh1_ref[0:1])
  o_ref[...] = _dot(f, wh2_ref[...]) + bh2_ref[0:1]


def _mlphead_call(node_h, a0, a1, w1, b18, w2, b28, g8, lb8, b3, pp, wp1p,
                  bp18, wp2, bp28, whb, wh1a, bh18, wh2p, bh28):
  full = lambda i: (0, 0)
  return pl.pallas_call(
      _mlphead_body,
      grid=(NB,),
      in_specs=_MLP_SPECS + [
          pl.BlockSpec((1, 1, 256), lambda i: (i, 0, 0)),
          pl.BlockSpec((128, 128), full),
          pl.BlockSpec((128, H), full),
          pl.BlockSpec((8, H), full),
          pl.BlockSpec((H, H), full),
          pl.BlockSpec((8, H), full),
          pl.BlockSpec((H, H), full),
          pl.BlockSpec((H, H), full),
          pl.BlockSpec((8, H), full),
          pl.BlockSpec((H, 128), full),
          pl.BlockSpec((8, 128), full),
      ],
      out_specs=pl.BlockSpec((256, 128), lambda i: (i, 0)),
      out_shape=jax.ShapeDtypeStruct((NP, 128), jnp.float32),
  )(node_h, a0, a1, w1, b18, w2, b28, g8, lb8, b3, pp, wp1p, bp18, wp2, bp28,
    whb, wh1a, bh18, wh2p, bh28)


HCH = CHUNKS // 2


def _sc_half(y_hbm, gidx_hbm, dst_hbm, zer_hbm, out_hbm,
             idx_v, dst_v, rows0_v, rows1_v, acc_sh, gs0, gs1, ss0, ss1, s):
  def gather(j, buf, sem):
    pltpu.async_copy(y_hbm.at[idx_v.at[j]], buf, sem)

  def scatter(j, buf, sem):
    pltpu.async_copy(buf, acc_sh.at[dst_v.at[j]], sem, add=True)

  def gather_wait(j, buf, sem):
    pltpu.make_async_copy(y_hbm.at[idx_v.at[j]], buf, sem).wait()

  def scatter_wait(j, buf, sem):
    pltpu.make_async_copy(buf, acc_sh.at[dst_v.at[j]], sem).wait()

  pltpu.sync_copy(gidx_hbm.at[s].at[pl.ds(0, HCH), :], idx_v)
  pltpu.sync_copy(dst_hbm.at[s].at[pl.ds(0, HCH), :], dst_v)
  gather(0, rows0_v, gs0)
  zslc = rows1_v.at[pl.ds(0, 64), :]
  pltpu.sync_copy(zer_hbm, zslc)
  for i in range(RPS // 64):
    pltpu.sync_copy(zslc, acc_sh.at[pl.ds(s * RPS + i * 64, 64), :])
  plsc.subcore_barrier()

  for h in range(2):

    def body(k, _):
      j0 = 2 * k
      j1 = j0 + 1

      @pl.when(k > 0)
      def _():
        scatter_wait(j1 - 2, rows1_v, ss1)

      gather(j1, rows1_v, gs1)
      gather_wait(j0, rows0_v, gs0)
      scatter(j0, rows0_v, ss0)

      @pl.when(k < HCH // 2 - 1)
      def _():
        scatter_wait(j0, rows0_v, ss0)
        gather(j0 + 2, rows0_v, gs0)

      gather_wait(j1, rows1_v, gs1)
      scatter(j1, rows1_v, ss1)
      return 0

    lax.fori_loop(0, HCH // 2, body, 0)
    scatter_wait(HCH - 2, rows0_v, ss0)
    scatter_wait(HCH - 1, rows1_v, ss1)
    if h == 0:
      pltpu.sync_copy(gidx_hbm.at[s].at[pl.ds(HCH, HCH), :], idx_v)
      pltpu.sync_copy(dst_hbm.at[s].at[pl.ds(HCH, HCH), :], dst_v)
      gather(0, rows0_v, gs0)

  plsc.subcore_barrier()
  pltpu.sync_copy(acc_sh.at[pl.ds(s * RPS, RPS), :],
                  out_hbm.at[pl.ds(s * RPS, RPS), :])


def _sc_body(y0_hbm, y1_hbm, gidx_hbm, dst_hbm, zer_hbm, out0_hbm, out1_hbm,
             idx_v, dst_v, rows0_v, rows1_v, acc_sh, gs0, gs1, ss0, ss1):
  c = lax.axis_index("c")
  s = lax.axis_index("s")
  args = (gidx_hbm, dst_hbm, zer_hbm)
  scratch = (idx_v, dst_v, rows0_v, rows1_v, acc_sh, gs0, gs1, ss0, ss1)

  @pl.when(c == 0)
  def _():
    _sc_half(y0_hbm, *args, out0_hbm, *scratch, s)

  @pl.when(c == 1)
  def _():
    _sc_half(y1_hbm, *args, out1_hbm, *scratch, s)


@functools.cache
def _sc_aggregate_fn():
  return pl.kernel(
      _sc_body,
      out_type=[jax.ShapeDtypeStruct((NP, HH), jnp.float32),
                jax.ShapeDtypeStruct((NP, HH), jnp.float32)],
      mesh=plsc.VectorSubcoreMesh(core_axis_name="c", subcore_axis_name="s",
                                  num_cores=2, num_subcores=NSUB),
      scratch_types=[
          pltpu.VMEM((HCH, EPC), jnp.int32),
          pltpu.VMEM((HCH, EPC), jnp.int32),
          pltpu.VMEM((EPC, HH), jnp.float32),
          pltpu.VMEM((EPC, HH), jnp.float32),
          pltpu.VMEM_SHARED((NP, HH), jnp.float32),
          pltpu.SemaphoreType.DMA,
          pltpu.SemaphoreType.DMA,
          pltpu.SemaphoreType.DMA,
          pltpu.SemaphoreType.DMA,
      ],
  )


def _sc_aggregate(y0, y1, gidx3, dst3, zer):
  return _sc_aggregate_fn()(y0, y1, gidx3, dst3, zer)


def _b8(v, w=H):
  return jnp.broadcast_to(jnp.reshape(v, (1, -1)), (8, w))


@jax.jit
def _run(z, x, edge_index, edge_type, batch, props, atom_emb, bond_emb, Wx,
         bx, lin_W, lin_b, mlp_W1, mlp_b1, mlp_W2, mlp_b2, ln_g, ln_b, Wp1,
         bp1, Wp2, bp2, Wh1, bh1, Wh2, bh2):
  f32 = jnp.float32
  zc = jnp.clip(z, 0, MAX_Z - 1).astype(jnp.int32)
  z3 = jnp.pad(zc, (0, NP - N)).reshape(NB, 1, 256)
  xp = jnp.pad(x.astype(f32), ((0, NP - N), (0, 122)))
  aep = jnp.pad(atom_emb.astype(f32), ((0, 128 - MAX_Z), (0, 0)))
  wxp = jnp.pad(Wx.astype(f32), ((0, 122), (0, 0)))

  srci = edge_index[0].astype(jnp.int32)
  dsti = edge_index[1].astype(jnp.int32)
  et = jnp.clip(edge_type, 0, NBT - 1).astype(jnp.int32)
  gidx = et * NP + srci
  gidx3 = jnp.pad(gidx, (0, EPAD - E)).reshape(NSUB, CHUNKS, EPC)
  dst3 = jnp.pad(dsti, (0, EPAD - E),
                 constant_values=DUMMY_DST).reshape(NSUB, CHUNKS, EPC)
  zer = jnp.zeros((64, HH), f32)

  bep = jnp.pad(bond_emb.astype(f32), ((0, 8 - NBT), (0, 0)))
  mw = lambda l: (mlp_W1[l], _b8(mlp_b1[l]), mlp_W2[l], _b8(mlp_b2[l]),
                  _b8(ln_g[l]), _b8(ln_b[l]))

  node_h, y0, y1 = _inityt_call(z3, xp, aep, wxp, _b8(bx), bep, lin_W[0],
                                _b8(lin_b[0]))
  a0, a1 = _sc_aggregate(y0.reshape(NBT * NP, HH), y1.reshape(NBT * NP, HH),
                         gidx3, dst3, zer)
  node_h, y0, y1 = _mlpyt_call(node_h, a0, a1, *mw(0), bep, lin_W[1],
                               _b8(lin_b[1]))
  a0, a1 = _sc_aggregate(y0.reshape(NBT * NP, HH), y1.reshape(NBT * NP, HH),
                         gidx3, dst3, zer)

  pp = jnp.pad(props.astype(f32), ((0, 128 - G), (0, 128 - P)))
  wp1p = jnp.pad(Wp1.astype(f32), ((0, 128 - P), (0, 0)))
  b3 = jnp.pad(batch.astype(jnp.int32), (0, NP - N)).reshape(NB, 1, 256)
  wh2p = jnp.pad(Wh2.astype(f32), ((0, 0), (0, 124)))
  bh2p = jnp.pad(bh2.astype(f32), (0, 124))
  out = _mlphead_call(node_h, a0, a1, *mw(1), b3, pp, wp1p, _b8(bp1), Wp2,
                      _b8(bp2), Wh1[H:], Wh1[:H], _b8(bh1), wh2p,
                      _b8(bh2p, 128))
  return out[:N, :4]


def kernel(z, x, edge_index, edge_type, batch, props, atom_emb, bond_emb, Wx,
           bx, lin_W, lin_b, mlp_W1, mlp_b1, mlp_W2, mlp_b2, ln_g, ln_b, Wp1,
           bp1, Wp2, bp2, Wh1, bh1, Wh2, bh2):
  return _run(z, x, edge_index, edge_type, batch, props, atom_emb, bond_emb,
              Wx, bx, lin_W, lin_b, mlp_W1, mlp_b1, mlp_W2, mlp_b2, ln_g,
              ln_b, Wp1, bp1, Wp2, bp2, Wh1, bh1, Wh2, bh2)


# speedup vs baseline: 1.0191x; 1.0191x over previous
import functools

import jax
import jax.numpy as jnp
from jax import lax
from jax.experimental import pallas as pl
from jax.experimental.pallas import tpu as pltpu
from jax.experimental.pallas import tpu_sc as plsc

N = 10000
E = 160000
H = 256
G = 64
P = 32
MAX_Z = 101
NBT = 5
L = 2

NP = 10240
NB = NP // 256
HH = H // 2
NSUB = 16
EPC = 128
CHUNKS = 80
EPAD = NSUB * CHUNKS * EPC
RPS = NP // NSUB
DUMMY_DST = N + 128

_HIGH = lax.Precision.HIGHEST


def _dot(a, b):
  return jnp.dot(a, b, preferred_element_type=jnp.float32)


def _dot_t(a, b):
  return lax.dot_general(a, b, (((0,), (0,)), ((), ())),
                         preferred_element_type=jnp.float32, precision=_HIGH)


def _ytab_part(nh, be, w, b, y0_ref, y1_ref):
  e5 = _dot(be, w) + b[0:1]
  for t in range(NBT):
    y = jax.nn.relu(nh + e5[t:t + 1])
    y0_ref[t] = y[:, :HH]
    y1_ref[t] = y[:, HH:]


def _mlp_part(nh_ref, a0_ref, a1_ref, w1_ref, b1_ref, w2_ref, b2_ref, g_ref,
              lb_ref):
  ag = jnp.concatenate([a0_ref[...], a1_ref[...]], axis=-1)
  h = nh_ref[...] + ag
  t1 = jax.nn.relu(_dot(h, w1_ref[...]) + b1_ref[0:1])
  t2 = _dot(t1, w2_ref[...]) + b2_ref[0:1]
  mu = jnp.mean(t2, axis=-1, keepdims=True)
  var = jnp.mean((t2 - mu) ** 2, axis=-1, keepdims=True)
  hn = (t2 - mu) / jnp.sqrt(var + 1e-5) * g_ref[0:1] + lb_ref[0:1]
  return jax.nn.relu(hn)


_YTAB_SPECS = [
    pl.BlockSpec((8, H), lambda i: (0, 0)),
    pl.BlockSpec((H, H), lambda i: (0, 0)),
    pl.BlockSpec((8, H), lambda i: (0, 0)),
]
_YTAB_OUT_SPECS = [
    pl.BlockSpec((NBT, 256, HH), lambda i: (0, i, 0)),
    pl.BlockSpec((NBT, 256, HH), lambda i: (0, i, 0)),
]
_YTAB_OUT_SHAPES = [
    jax.ShapeDtypeStruct((NBT, NP, HH), jnp.float32),
    jax.ShapeDtypeStruct((NBT, NP, HH), jnp.float32),
]


def _inityt_body(z_ref, x_ref, ae_ref, wx_ref, bx_ref, be_ref, w_ref, b_ref,
                 nh_ref, y0_ref, y1_ref):
  ohT = (lax.broadcasted_iota(jnp.int32, (128, 256), 0)
         == z_ref[0]).astype(jnp.float32)
  nh = _dot_t(ohT, ae_ref[...]) + _dot(x_ref[...], wx_ref[...]) + bx_ref[0:1]
  nh = jax.nn.relu(nh)
  nh_ref[...] = nh
  _ytab_part(nh, be_ref[...], w_ref[...], b_ref[...], y0_ref, y1_ref)


def _inityt_call(z3, xp, aep, wxp, bx8, bep, w, b8):
  return pl.pallas_call(
      _inityt_body,
      grid=(NB,),
      in_specs=[
          pl.BlockSpec((1, 1, 256), lambda i: (i, 0, 0)),
          pl.BlockSpec((256, 128), lambda i: (i, 0)),
          pl.BlockSpec((128, H), lambda i: (0, 0)),
          pl.BlockSpec((128, H), lambda i: (0, 0)),
          pl.BlockSpec((8, H), lambda i: (0, 0)),
      ] + _YTAB_SPECS,
      out_specs=[pl.BlockSpec((256, H), lambda i: (i, 0))] + _YTAB_OUT_SPECS,
      out_shape=[jax.ShapeDtypeStruct((NP, H), jnp.float32)]
      + _YTAB_OUT_SHAPES,
  )(z3, xp, aep, wxp, bx8, bep, w, b8)


_MLP_SPECS = [
    pl.BlockSpec((256, H), lambda i: (i, 0)),
    pl.BlockSpec((256, HH), lambda i: (i, 0)),
    pl.BlockSpec((256, HH), lambda i: (i, 0)),
    pl.BlockSpec((H, H), lambda i: (0, 0)),
    pl.BlockSpec((8, H), lambda i: (0, 0)),
    pl.BlockSpec((H, H), lambda i: (0, 0)),
    pl.BlockSpec((8, H), lambda i: (0, 0)),
    pl.BlockSpec((8, H), lambda i: (0, 0)),
    pl.BlockSpec((8, H), lambda i: (0, 0)),
]


def _mlpyt_body(nh_ref, a0_ref, a1_ref, w1_ref, b1_ref, w2_ref, b2_ref,
                g_ref, lb_ref, be_ref, w_ref, b_ref, nh_o, y0_ref, y1_ref):
  nh = _mlp_part(nh_ref, a0_ref, a1_ref, w1_ref, b1_ref, w2_ref, b2_ref,
                 g_ref, lb_ref)
  nh_o[...] = nh
  _ytab_part(nh, be_ref[...], w_ref[...], b_ref[...], y0_ref, y1_ref)


def _mlpyt_call(node_h, a0, a1, w1, b18, w2, b28, g8, lb8, bep, w, b8):
  return pl.pallas_call(
      _mlpyt_body,
      grid=(NB,),
      in_specs=_MLP_SPECS + _YTAB_SPECS,
      out_specs=[pl.BlockSpec((256, H), lambda i: (i, 0))] + _YTAB_OUT_SPECS,
      out_shape=[jax.ShapeDtypeStruct((NP, H), jnp.float32)]
      + _YTAB_OUT_SHAPES,
  )(node_h, a0, a1, w1, b18, w2, b28, g8, lb8, bep, w, b8)


def _ctx_body(pp_ref, w1_ref, b1_ref, w2_ref, b2_ref, wh_ref, o_ref):
  c1 = jax.nn.relu(_dot(pp_ref[...], w1_ref[...]) + b1_ref[0:1])
  ctx = _dot(c1, w2_ref[...]) + b2_ref[0:1]
  o_ref[...] = _dot(ctx, wh_ref[...])


def _ctx_call(pp, wp1p, bp18, wp2, bp28, wh1b):
  return pl.pallas_call(
      _ctx_body,
      grid=(),
      out_shape=jax.ShapeDtypeStruct((G, H), jnp.float32),
  )(pp, wp1p, bp18, wp2, bp28, wh1b)


def _mlphead_body(nh_ref, a0_ref, a1_ref, w1_ref, b1_ref, w2_ref, b2_ref,
                  g_ref, lb_ref, b3_ref, cw_ref, wh1_ref, bh1_ref, wh2_ref,
                  bh2_ref, o_ref):
  nh = _mlp_part(nh_ref, a0_ref, a1_ref, w1_ref, b1_ref, w2_ref, b2_ref,
                 g_ref, lb_ref)
  ohT = (lax.broadcasted_iota(jnp.int32, (128, 256), 0)
         == b3_ref[0]).astype(jnp.float32)
  f = jax.nn.relu(_dot(nh, wh1_ref[...]) + _dot_t(ohT, cw_ref[...])
                  + bh1_ref[0:1])
  o_ref[...] = _dot(f, wh2_ref[...]) + bh2_ref[0:1]


def _mlphead_call(node_h, a0, a1, w1, b18, w2, b28, g8, lb8, b3, cwp, wh1a,
                  bh18, wh2p, bh28):
  full = lambda i: (0, 0)
  return pl.pallas_call(
      _mlphead_body,
      grid=(NB,),
      in_specs=_MLP_SPECS + [
          pl.BlockSpec((1, 1, 256), lambda i: (i, 0, 0)),
          pl.BlockSpec((128, H), full),
          pl.BlockSpec((H, H), full),
          pl.BlockSpec((8, H), full),
          pl.BlockSpec((H, 128), full),
          pl.BlockSpec((8, 128), full),
      ],
      out_specs=pl.BlockSpec((256, 128), lambda i: (i, 0)),
      out_shape=jax.ShapeDtypeStruct((NP, 128), jnp.float32),
  )(node_h, a0, a1, w1, b18, w2, b28, g8, lb8, b3, cwp, wh1a, bh18, wh2p,
    bh28)


HCH = CHUNKS // 2


def _sc_half(y_hbm, gidx_hbm, dst_hbm, zer_hbm, out_hbm,
             idx_v, dst_v, rows0_v, rows1_v, acc_sh, gs0, gs1, ss0, ss1, s):
  def gather(j, buf, sem):
    pltpu.async_copy(y_hbm.at[idx_v.at[j]], buf, sem)

  def scatter(j, buf, sem):
    pltpu.async_copy(buf, acc_sh.at[dst_v.at[j]], sem, add=True)

  def gather_wait(j, buf, sem):
    pltpu.make_async_copy(y_hbm.at[idx_v.at[j]], buf, sem).wait()

  def scatter_wait(j, buf, sem):
    pltpu.make_async_copy(buf, acc_sh.at[dst_v.at[j]], sem).wait()

  pltpu.sync_copy(gidx_hbm.at[s].at[pl.ds(0, HCH), :], idx_v)
  pltpu.sync_copy(dst_hbm.at[s].at[pl.ds(0, HCH), :], dst_v)
  gather(0, rows0_v, gs0)
  zslc = rows1_v.at[pl.ds(0, 64), :]
  pltpu.sync_copy(zer_hbm, zslc)
  for i in range(RPS // 64):
    pltpu.sync_copy(zslc, acc_sh.at[pl.ds(s * RPS + i * 64, 64), :])
  plsc.subcore_barrier()

  for h in range(2):

    def body(k, _):
      j0 = 2 * k
      j1 = j0 + 1

      @pl.when(k > 0)
      def _():
        scatter_wait(j1 - 2, rows1_v, ss1)

      gather(j1, rows1_v, gs1)
      gather_wait(j0, rows0_v, gs0)
      scatter(j0, rows0_v, ss0)

      @pl.when(k < HCH // 2 - 1)
      def _():
        scatter_wait(j0, rows0_v, ss0)
        gather(j0 + 2, rows0_v, gs0)

      gather_wait(j1, rows1_v, gs1)
      scatter(j1, rows1_v, ss1)
      return 0

    lax.fori_loop(0, HCH // 2, body, 0)
    scatter_wait(HCH - 2, rows0_v, ss0)
    scatter_wait(HCH - 1, rows1_v, ss1)
    if h == 0:
      pltpu.sync_copy(gidx_hbm.at[s].at[pl.ds(HCH, HCH), :], idx_v)
      pltpu.sync_copy(dst_hbm.at[s].at[pl.ds(HCH, HCH), :], dst_v)
      gather(0, rows0_v, gs0)

  plsc.subcore_barrier()
  pltpu.sync_copy(acc_sh.at[pl.ds(s * RPS, RPS), :],
                  out_hbm.at[pl.ds(s * RPS, RPS), :])


def _sc_body(y0_hbm, y1_hbm, gidx_hbm, dst_hbm, zer_hbm, out0_hbm, out1_hbm,
             idx_v, dst_v, rows0_v, rows1_v, acc_sh, gs0, gs1, ss0, ss1):
  c = lax.axis_index("c")
  s = lax.axis_index("s")
  args = (gidx_hbm, dst_hbm, zer_hbm)
  scratch = (idx_v, dst_v, rows0_v, rows1_v, acc_sh, gs0, gs1, ss0, ss1)

  @pl.when(c == 0)
  def _():
    _sc_half(y0_hbm, *args, out0_hbm, *scratch, s)

  @pl.when(c == 1)
  def _():
    _sc_half(y1_hbm, *args, out1_hbm, *scratch, s)


@functools.cache
def _sc_aggregate_fn():
  return pl.kernel(
      _sc_body,
      out_type=[jax.ShapeDtypeStruct((NP, HH), jnp.float32),
                jax.ShapeDtypeStruct((NP, HH), jnp.float32)],
      mesh=plsc.VectorSubcoreMesh(core_axis_name="c", subcore_axis_name="s",
                                  num_cores=2, num_subcores=NSUB),
      scratch_types=[
          pltpu.VMEM((HCH, EPC), jnp.int32),
          pltpu.VMEM((HCH, EPC), jnp.int32),
          pltpu.VMEM((EPC, HH), jnp.float32),
          pltpu.VMEM((EPC, HH), jnp.float32),
          pltpu.VMEM_SHARED((NP, HH), jnp.float32),
          pltpu.SemaphoreType.DMA,
          pltpu.SemaphoreType.DMA,
          pltpu.SemaphoreType.DMA,
          pltpu.SemaphoreType.DMA,
      ],
  )


def _sc_aggregate(y0, y1, gidx3, dst3, zer):
  return _sc_aggregate_fn()(y0, y1, gidx3, dst3, zer)


def _b8(v, w=H):
  return jnp.broadcast_to(jnp.reshape(v, (1, -1)), (8, w))


@jax.jit
def _run(z, x, edge_index, edge_type, batch, props, atom_emb, bond_emb, Wx,
         bx, lin_W, lin_b, mlp_W1, mlp_b1, mlp_W2, mlp_b2, ln_g, ln_b, Wp1,
         bp1, Wp2, bp2, Wh1, bh1, Wh2, bh2):
  f32 = jnp.float32
  zc = jnp.clip(z, 0, MAX_Z - 1).astype(jnp.int32)
  z3 = jnp.pad(zc, (0, NP - N)).reshape(NB, 1, 256)
  xp = jnp.pad(x.astype(f32), ((0, NP - N), (0, 122)))
  aep = jnp.pad(atom_emb.astype(f32), ((0, 128 - MAX_Z), (0, 0)))
  wxp = jnp.pad(Wx.astype(f32), ((0, 122), (0, 0)))

  srci = edge_index[0].astype(jnp.int32)
  dsti = edge_index[1].astype(jnp.int32)
  et = jnp.clip(edge_type, 0, NBT - 1).astype(jnp.int32)
  gidx = et * NP + srci
  gidx3 = jnp.pad(gidx, (0, EPAD - E)).reshape(NSUB, CHUNKS, EPC)
  dst3 = jnp.pad(dsti, (0, EPAD - E),
                 constant_values=DUMMY_DST).reshape(NSUB, CHUNKS, EPC)
  zer = jnp.zeros((64, HH), f32)

  bep = jnp.pad(bond_emb.astype(f32), ((0, 8 - NBT), (0, 0)))
  mw = lambda l: (mlp_W1[l], _b8(mlp_b1[l]), mlp_W2[l], _b8(mlp_b2[l]),
                  _b8(ln_g[l]), _b8(ln_b[l]))

  node_h, y0, y1 = _inityt_call(z3, xp, aep, wxp, _b8(bx), bep, lin_W[0],
                                _b8(lin_b[0]))
  a0, a1 = _sc_aggregate(y0.reshape(NBT * NP, HH), y1.reshape(NBT * NP, HH),
                         gidx3, dst3, zer)
  node_h, y0, y1 = _mlpyt_call(node_h, a0, a1, *mw(0), bep, lin_W[1],
                               _b8(lin_b[1]))
  a0, a1 = _sc_aggregate(y0.reshape(NBT * NP, HH), y1.reshape(NBT * NP, HH),
                         gidx3, dst3, zer)

  pp = jnp.pad(props.astype(f32), ((0, 0), (0, 128 - P)))
  wp1p = jnp.pad(Wp1.astype(f32), ((0, 128 - P), (0, 0)))
  ctxw = _ctx_call(pp, wp1p, _b8(bp1), Wp2, _b8(bp2), Wh1[H:])
  cwp = jnp.pad(ctxw, ((0, 128 - G), (0, 0)))
  b3 = jnp.pad(batch.astype(jnp.int32), (0, NP - N)).reshape(NB, 1, 256)
  wh2p = jnp.pad(Wh2.astype(f32), ((0, 0), (0, 124)))
  bh2p = jnp.pad(bh2.astype(f32), (0, 124))
  out = _mlphead_call(node_h, a0, a1, *mw(1), b3, cwp, Wh1[:H], _b8(bh1),
                      wh2p, _b8(bh2p, 128))
  return out[:N, :4]


def kernel(z, x, edge_index, edge_type, batch, props, atom_emb, bond_emb, Wx,
           bx, lin_W, lin_b, mlp_W1, mlp_b1, mlp_W2, mlp_b2, ln_g, ln_b, Wp1,
           bp1, Wp2, bp2, Wh1, bh1, Wh2, bh2):
  return _run(z, x, edge_index, edge_type, batch, props, atom_emb, bond_emb,
              Wx, bx, lin_W, lin_b, mlp_W1, mlp_b1, mlp_W2, mlp_b2, ln_g,
              ln_b, Wp1, bp1, Wp2, bp2, Wh1, bh1, Wh2, bh2)
